# Initial kernel scaffold; baseline (speedup 1.0000x reference)
import jax
import jax.numpy as jnp
from jax.experimental import pallas as pl


def kernel(node_attr, edge_index, edge_attr, edge_sh, fc_w1, fc_b1, fc_w2, fc_b2, bn_gamma, bn_beta):
    raise NotImplementedError("write your pallas kernel here")


# speedup vs baseline: 2.2207x; 2.2207x over previous
import functools

import jax
import jax.numpy as jnp
from jax import lax
from jax.experimental import pallas as pl
from jax.experimental.pallas import tpu as pltpu
from jax.experimental.pallas import tpu_sc as plsc

_N = 10000
_E = 160000
_IN = 32
_OUT = 32
_NEF = 16
_HID = 64
_EPS = 1e-5

_NW = 32
_CHUNK = 128
_NCH = 40
_EPW = _CHUNK * _NCH
_EP = _EPW * _NW
_NBUF = 4
_SBUF = 2
_NP = 10240
_ROWS_PER_SUB = _NP // 16
_TCB = 512


def _gather_body(node4_hbm, dst4_hbm, x_hbm, idx_v, rows_v, sem):
    c = lax.axis_index("c")
    s = lax.axis_index("s")
    wid = s * 2 + c
    pltpu.sync_copy(dst4_hbm.at[wid], idx_v)
    base = wid * _EPW
    for g in range(_NCH // _NBUF):
        cps = []
        for b in range(_NBUF):
            j = g * _NBUF + b
            cps.append(pltpu.async_copy(node4_hbm.at[idx_v.at[j]], rows_v.at[b], sem))
        for b in range(_NBUF):
            j = g * _NBUF + b
            cps[b].wait()
            pltpu.sync_copy(rows_v.at[b], x_hbm.at[pl.ds(base + j * _CHUNK, _CHUNK)])


def _scatter_body(tpc_hbm, src_hbm, zinit_hbm, out_hbm, idx_v, rows_v, acc_sh, sem):
    c = lax.axis_index("c")
    s = lax.axis_index("s")
    wid = s * 2 + c
    pltpu.sync_copy(zinit_hbm.at[pl.ds(s * _ROWS_PER_SUB, _ROWS_PER_SUB)],
                    acc_sh.at[pl.ds(s * _ROWS_PER_SUB, _ROWS_PER_SUB)])
    pltpu.sync_copy(src_hbm.at[wid], idx_v)
    plsc.subcore_barrier()
    base = wid * _EPW
    for g in range(_NCH // _SBUF):
        cps = []
        for b in range(_SBUF):
            j = g * _SBUF + b
            cps.append(pltpu.async_copy(
                tpc_hbm.at[pl.ds(base + j * _CHUNK, _CHUNK)], rows_v.at[b], sem))
        for b in range(_SBUF):
            j = g * _SBUF + b
            cps[b].wait()
            pltpu.sync_copy(rows_v.at[b], acc_sh.at[idx_v.at[j]], add=True)
    plsc.subcore_barrier()
    pltpu.sync_copy(acc_sh.at[pl.ds(s * _ROWS_PER_SUB, _ROWS_PER_SUB)],
                    out_hbm.at[c, pl.ds(s * _ROWS_PER_SUB, _ROWS_PER_SUB)])


def _tc_body(ea_ref, sh_ref, off_ref, xq_ref, w1_ref, b1_ref, w2_ref, b2_ref,
             rep_ref, out_ref):
    xq = xq_ref[...]
    off = off_ref[...]
    x = ((off == 0) * xq[:, 0:32] + (off == 1) * xq[:, 32:64]
         + (off == 2) * xq[:, 64:96] + (off == 3) * xq[:, 96:128])
    h = jnp.maximum(
        jnp.dot(ea_ref[...], w1_ref[...], preferred_element_type=jnp.float32)
        + b1_ref[...], 0.0)
    wb = jnp.dot(h, w2_ref[...], preferred_element_type=jnp.float32) + b2_ref[...]
    xf = jnp.dot(x, rep_ref[...], preferred_element_type=jnp.float32)
    p = xf * wb
    p = p[:, :512] + p[:, 512:]
    p = p[:, :256] + p[:, 256:]
    p = p[:, :128] + p[:, 128:]
    p = p[:, :64] + p[:, 64:]
    p = p[:, :32] + p[:, 32:]
    tp = p * sh_ref[...]
    i = pl.program_id(0)
    valid = ((lax.broadcasted_iota(jnp.int32, (_TCB, 16), 0) + i * _TCB) < _E
             ).astype(jnp.float32)
    out_ref[...] = jnp.concatenate([tp, valid, jnp.zeros((_TCB, 80), jnp.float32)], axis=1)


def _fin_body(parts_ref, node_ref, gamma_ref, beta_ref, out_ref):
    sums = parts_ref[0, :_N, :_OUT] + parts_ref[1, :_N, :_OUT]
    cnt = parts_ref[0, :_N, _OUT:_OUT + 1] + parts_ref[1, :_N, _OUT:_OUT + 1]
    o = sums / jnp.maximum(cnt, 1.0) + node_ref[...]
    mu = jnp.mean(o, axis=0, keepdims=True)
    var = jnp.mean((o - mu) ** 2, axis=0, keepdims=True)
    out_ref[...] = (o - mu) * lax.rsqrt(var + _EPS) * gamma_ref[...] + beta_ref[...]


_sc_mesh = plsc.VectorSubcoreMesh(core_axis_name="c", subcore_axis_name="s")

_gather_call = functools.partial(
    pl.kernel,
    out_type=jax.ShapeDtypeStruct((_EP, 128), jnp.float32),
    mesh=_sc_mesh,
    scratch_types=[
        pltpu.VMEM((_NCH, _CHUNK), jnp.int32),
        pltpu.VMEM((_NBUF, _CHUNK, 128), jnp.float32),
        pltpu.SemaphoreType.DMA,
    ],
)(_gather_body)

_scatter_call = functools.partial(
    pl.kernel,
    out_type=jax.ShapeDtypeStruct((2, _NP, 128), jnp.float32),
    mesh=_sc_mesh,
    scratch_types=[
        pltpu.VMEM((_NCH, _CHUNK), jnp.int32),
        pltpu.VMEM((_SBUF, _CHUNK, 128), jnp.float32),
        pltpu.VMEM_SHARED((_NP, 128), jnp.float32),
        pltpu.SemaphoreType.DMA,
    ],
)(_scatter_body)


def kernel(node_attr, edge_index, edge_attr, edge_sh, fc_w1, fc_b1, fc_w2, fc_b2,
           bn_gamma, bn_beta):
    padn = _EP - _E
    dst = jnp.pad(edge_index[1], (0, padn))
    dst4 = (dst // 4).reshape(_NW, _NCH, _CHUNK)
    off = (dst % 4).astype(jnp.int32).reshape(_EP, 1)
    src = jnp.pad(edge_index[0], (0, padn)).reshape(_NW, _NCH, _CHUNK)
    ea_p = jnp.pad(edge_attr, ((0, padn), (0, 0)))
    sh_p = jnp.pad(edge_sh * (1.0 / jnp.sqrt(jnp.float32(_IN))), ((0, padn), (0, 0)))
    rep = jnp.repeat(jnp.eye(_IN, dtype=jnp.float32), _OUT, axis=1)
    zinit = jnp.zeros((_NP, 128), jnp.float32)
    node4 = node_attr.reshape(_N // 4, 128)

    xq = _gather_call(node4, dst4)

    tpc = pl.pallas_call(
        _tc_body,
        grid=(_EP // _TCB,),
        in_specs=[
            pl.BlockSpec((_TCB, _NEF), lambda i: (i, 0)),
            pl.BlockSpec((_TCB, 1), lambda i: (i, 0)),
            pl.BlockSpec((_TCB, 1), lambda i: (i, 0)),
            pl.BlockSpec((_TCB, 128), lambda i: (i, 0)),
            pl.BlockSpec((_NEF, _HID), lambda i: (0, 0)),
            pl.BlockSpec((1, _HID), lambda i: (0, 0)),
            pl.BlockSpec((_HID, _IN * _OUT), lambda i: (0, 0)),
            pl.BlockSpec((1, _IN * _OUT), lambda i: (0, 0)),
            pl.BlockSpec((_IN, _IN * _OUT), lambda i: (0, 0)),
        ],
        out_specs=pl.BlockSpec((_TCB, 128), lambda i: (i, 0)),
        out_shape=jax.ShapeDtypeStruct((_EP, 128), jnp.float32),
    )(ea_p, sh_p, off, xq, fc_w1, fc_b1.reshape(1, _HID), fc_w2,
      fc_b2.reshape(1, _IN * _OUT), rep)

    parts = _scatter_call(tpc, src, zinit)

    out = pl.pallas_call(
        _fin_body,
        out_shape=jax.ShapeDtypeStruct((_N, _OUT), jnp.float32),
    )(parts, node_attr, bn_gamma.reshape(1, _OUT), bn_beta.reshape(1, _OUT))
    return out
